# Initial kernel scaffold; baseline (speedup 1.0000x reference)
import jax
import jax.numpy as jnp
from jax.experimental import pallas as pl


def kernel(ctx_bits, mapping, target_vals, ram):
    raise NotImplementedError("write your pallas kernel here")


# speedup vs baseline: 66.2973x; 66.2973x over previous
import functools

import jax
import jax.numpy as jnp
from jax import lax
from jax.experimental import pallas as pl
from jax.experimental.pallas import tpu as pltpu
from jax.experimental.pallas import tpu_sc as plsc

NUM_NEURONS = 8
N_BITS = 10
RAM_SIZE = 1 << N_BITS
TOTAL_BITS = 32
TAB = NUM_NEURONS * RAM_SIZE

NC = 2
NS = 16
NW = NC * NS


def _tc_addr_body(ctx_ref, w_ref, out_ref):
    x = ctx_ref[...].astype(jnp.float32)
    y = jnp.dot(x, w_ref[...], preferred_element_type=jnp.float32)
    col = lax.broadcasted_iota(jnp.int32, y.shape, 1)
    out_ref[...] = y.astype(jnp.int32) + (col & 7) * RAM_SIZE


def _tc_addresses(ctx2, w4, rows, rb):
    grid = (rows // rb,)
    return pl.pallas_call(
        _tc_addr_body,
        grid=grid,
        in_specs=[
            pl.BlockSpec((rb, 4 * TOTAL_BITS), lambda i: (i, 0)),
            pl.BlockSpec((4 * TOTAL_BITS, 4 * NUM_NEURONS), lambda i: (0, 0)),
        ],
        out_specs=pl.BlockSpec((rb, 4 * NUM_NEURONS), lambda i: (i, 0)),
        out_shape=jax.ShapeDtypeStruct((rows, 4 * NUM_NEURONS), jnp.int32),
    )(ctx2, w4)


def _sc_scatter_kernel(e_total, win_e):
    chunk = e_total // NW
    n_win = chunk // win_e
    iters = win_e // 16
    mesh = plsc.VectorSubcoreMesh(core_axis_name="c", subcore_axis_name="s")

    @functools.partial(
        pl.kernel,
        out_type=jax.ShapeDtypeStruct((NW, TAB), jnp.int32),
        mesh=mesh,
        compiler_params=pltpu.CompilerParams(needs_layout_passes=False),
        scratch_types=[
            pltpu.VMEM((win_e,), jnp.int32),
            pltpu.VMEM((TAB,), jnp.int32),
        ],
    )
    def k(addr_hbm, tabs_hbm, awin, tab):
        wid = lax.axis_index("s") * NC + lax.axis_index("c")
        base = wid * chunk
        io16 = lax.iota(jnp.int32, 16)

        def init(i, c):
            tab[pl.ds(i * 16, 16)] = jnp.full((16,), -1, jnp.int32)
            return c

        lax.fori_loop(0, TAB // 16, init, 0)

        for w in range(n_win):
            wbase = base + w * win_e
            pltpu.sync_copy(addr_hbm.at[pl.ds(wbase, win_e)], awin)

            def body(i, c, wbase=wbase):
                idx = awin[pl.ds(i * 16, 16)]
                b = lax.shift_right_logical(wbase + i * 16 + io16, 3)
                plsc.store_scatter(tab, [idx], b)
                return c

            lax.fori_loop(0, iters, body, 0)

        pltpu.sync_copy(tab, tabs_hbm.at[wid])

    return k


def _sc_merge_kernel(e_total):
    mesh = plsc.VectorSubcoreMesh(core_axis_name="c", subcore_axis_name="s")
    per_w = TAB // NW

    @functools.partial(
        pl.kernel,
        out_type=jax.ShapeDtypeStruct((TAB,), jnp.float32),
        mesh=mesh,
        compiler_params=pltpu.CompilerParams(needs_layout_passes=False),
        scratch_types=[
            pltpu.VMEM((NW, 128), jnp.int32),
            pltpu.VMEM((128,), jnp.int32),
            pltpu.VMEM((128,), jnp.float32),
            pltpu.SemaphoreType.DMA,
        ],
    )
    def k(tabs_hbm, tvals_hbm, vtab_hbm, tloc, gidx, gval, sem):
        wid = lax.axis_index("s") * NC + lax.axis_index("c")
        neuron = wid // (RAM_SIZE // per_w)
        for p in range(per_w // 128):
            col0 = wid * per_w + p * 128
            pltpu.sync_copy(tabs_hbm.at[:, pl.ds(col0, 128)], tloc)
            for g in range(8):
                m = tloc[0, pl.ds(g * 16, 16)]
                for r in range(1, NW):
                    m = jnp.maximum(m, tloc[r, pl.ds(g * 16, 16)])
                gi = m * NUM_NEURONS + neuron
                gidx[pl.ds(g * 16, 16)] = jnp.where(m < 0, 0, gi)
            pltpu.async_copy(tvals_hbm.at[gidx], gval, sem).wait()
            pltpu.sync_copy(gval, vtab_hbm.at[pl.ds(col0, 128)])

    return k


def _sc_gather_kernel(e_total, win_e):
    chunk = e_total // NW
    n_win = chunk // win_e
    iters = win_e // 16
    mesh = plsc.VectorSubcoreMesh(core_axis_name="c", subcore_axis_name="s")

    @functools.partial(
        pl.kernel,
        out_type=jax.ShapeDtypeStruct((e_total,), jnp.float32),
        mesh=mesh,
        compiler_params=pltpu.CompilerParams(needs_layout_passes=False),
        scratch_types=[
            pltpu.VMEM((TAB,), jnp.float32),
            pltpu.VMEM((win_e,), jnp.int32),
            pltpu.VMEM((win_e,), jnp.float32),
        ],
    )
    def k(addr_hbm, vtab_hbm, out_hbm, vtab, awin, owin):
        wid = lax.axis_index("s") * NC + lax.axis_index("c")
        base = wid * chunk
        pltpu.sync_copy(vtab_hbm, vtab)
        for w in range(n_win):
            wbase = base + w * win_e
            pltpu.sync_copy(addr_hbm.at[pl.ds(wbase, win_e)], awin)

            def body(i, c):
                idx = awin[pl.ds(i * 16, 16)]
                owin[pl.ds(i * 16, 16)] = plsc.load_gather(vtab, [idx])
                return c

            lax.fori_loop(0, iters, body, 0)
            pltpu.sync_copy(owin, out_hbm.at[pl.ds(wbase, win_e)])

    return k


def kernel(ctx_bits, mapping, target_vals, ram):
    del ram
    b_sz = ctx_bits.shape[0]
    e_total = b_sz * NUM_NEURONS

    powers = (1 << jnp.arange(N_BITS, dtype=jnp.int32))
    w = jnp.sum(
        (mapping[:, :, None] == jnp.arange(TOTAL_BITS, dtype=jnp.int32)[None, None, :])
        * powers[None, :, None],
        axis=1,
    ).T.astype(jnp.float32)
    w4 = jnp.kron(jnp.eye(4, dtype=jnp.float32), w)

    rows = b_sz // 4
    ctx2 = ctx_bits.reshape(rows, 4 * TOTAL_BITS)
    addr_flat = _tc_addresses(ctx2, w4, rows, 4096).reshape(-1)

    tvals_flat = target_vals.reshape(-1)
    tabs = _sc_scatter_kernel(e_total, 16384)(addr_flat)
    vtab = _sc_merge_kernel(e_total)(tabs, tvals_flat)
    out_flat = _sc_gather_kernel(e_total, 16384)(addr_flat, vtab)
    return out_flat.reshape(b_sz, NUM_NEURONS)


# speedup vs baseline: 414.7585x; 6.2560x over previous
import functools

import jax
import jax.numpy as jnp
from jax import lax
from jax.experimental import pallas as pl
from jax.experimental.pallas import tpu as pltpu
from jax.experimental.pallas import tpu_sc as plsc

NUM_NEURONS = 8
N_BITS = 10
RAM_SIZE = 1 << N_BITS
TOTAL_BITS = 32
TAB = NUM_NEURONS * RAM_SIZE

NC = 2
NS = 16
NW = NC * NS


def _tc_addr_body(w_ref, ctx_ref, out_ref):
    x = ctx_ref[...].astype(jnp.float32)
    y = jnp.dot(w_ref[...], x, preferred_element_type=jnp.float32)
    row = lax.broadcasted_iota(jnp.int32, y.shape, 0)
    out_ref[...] = y.astype(jnp.int32) + row * RAM_SIZE


def _tc_addresses(wt, ctx_t, b_sz, cb):
    grid = (b_sz // cb,)
    return pl.pallas_call(
        _tc_addr_body,
        grid=grid,
        in_specs=[
            pl.BlockSpec((NUM_NEURONS, TOTAL_BITS), lambda i: (0, 0)),
            pl.BlockSpec((TOTAL_BITS, cb), lambda i: (0, i)),
        ],
        out_specs=pl.BlockSpec((NUM_NEURONS, cb), lambda i: (0, i)),
        out_shape=jax.ShapeDtypeStruct((NUM_NEURONS, b_sz), jnp.int32),
    )(wt, ctx_t)


def _sc_scatter_kernel(b_sz, win_e):
    chunk = b_sz // 4
    n_win = chunk // win_e
    iters = win_e // 16
    mesh = plsc.VectorSubcoreMesh(core_axis_name="c", subcore_axis_name="s")

    @functools.partial(
        pl.kernel,
        out_type=jax.ShapeDtypeStruct((NW, TAB), jnp.int32),
        mesh=mesh,
        compiler_params=pltpu.CompilerParams(needs_layout_passes=False),
        scratch_types=[
            pltpu.VMEM((2, win_e), jnp.int32),
            pltpu.VMEM((TAB,), jnp.int32),
            pltpu.SemaphoreType.DMA,
            pltpu.SemaphoreType.DMA,
        ],
    )
    def k(addr_hbm, tabs_hbm, awin, tab, sem0, sem1):
        wid = lax.axis_index("s") * NC + lax.axis_index("c")
        neuron = wid // 4
        b0 = (wid % 4) * chunk
        ebase = neuron * b_sz + b0
        io16 = lax.iota(jnp.int32, 16)
        sems = (sem0, sem1)

        def init(i, c):
            tab[pl.ds(i * 16, 16)] = jnp.full((16,), -1, jnp.int32)
            return c

        lax.fori_loop(0, TAB // 16, init, 0)

        descs = [None, None]
        descs[0] = pltpu.async_copy(
            addr_hbm.at[neuron, pl.ds(b0, win_e)], awin.at[0], sems[0])
        for w in range(n_win):
            cur = w & 1
            if w + 1 < n_win:
                nxt = (w + 1) & 1
                descs[nxt] = pltpu.async_copy(
                    addr_hbm.at[neuron, pl.ds(b0 + (w + 1) * win_e, win_e)],
                    awin.at[nxt], sems[nxt])
            descs[cur].wait()
            wbase = ebase + w * win_e

            def body(i, c, wbase=wbase, cur=cur):
                offs = [i * 128 + u * 16 for u in range(8)]
                idxv = [awin[cur, pl.ds(o, 16)] for o in offs]
                evs = [wbase + o + io16 for o in offs]
                for u in range(8):
                    plsc.store_scatter(tab, [idxv[u]], evs[u])
                return c

            lax.fori_loop(0, iters // 8, body, 0)

        pltpu.sync_copy(tab, tabs_hbm.at[wid])

    return k


def _sc_merge_kernel():
    mesh = plsc.VectorSubcoreMesh(core_axis_name="c", subcore_axis_name="s")
    per_w = TAB // NW

    @functools.partial(
        pl.kernel,
        out_type=jax.ShapeDtypeStruct((TAB,), jnp.float32),
        mesh=mesh,
        compiler_params=pltpu.CompilerParams(needs_layout_passes=False),
        scratch_types=[
            pltpu.VMEM((NW, 128), jnp.int32),
            pltpu.VMEM((128,), jnp.int32),
            pltpu.VMEM((128,), jnp.float32),
            pltpu.SemaphoreType.DMA,
        ],
    )
    def k(tabs_hbm, tvals_hbm, vtab_hbm, tloc, gidx, gval, sem):
        wid = lax.axis_index("s") * NC + lax.axis_index("c")
        for p in range(per_w // 128):
            col0 = wid * per_w + p * 128
            pltpu.sync_copy(tabs_hbm.at[:, pl.ds(col0, 128)], tloc)
            for g in range(8):
                m = tloc[0, pl.ds(g * 16, 16)]
                for r in range(1, NW):
                    m = jnp.maximum(m, tloc[r, pl.ds(g * 16, 16)])
                gidx[pl.ds(g * 16, 16)] = jnp.maximum(m, 0)
            pltpu.async_copy(tvals_hbm.at[gidx], gval, sem).wait()
            pltpu.sync_copy(gval, vtab_hbm.at[pl.ds(col0, 128)])

    return k


def _sc_gather_kernel(b_sz, win_e):
    chunk = b_sz // 4
    n_win = chunk // win_e
    iters = win_e // 16
    mesh = plsc.VectorSubcoreMesh(core_axis_name="c", subcore_axis_name="s")

    @functools.partial(
        pl.kernel,
        out_type=jax.ShapeDtypeStruct((NUM_NEURONS, b_sz), jnp.float32),
        mesh=mesh,
        compiler_params=pltpu.CompilerParams(needs_layout_passes=False),
        scratch_types=[
            pltpu.VMEM((TAB,), jnp.float32),
            pltpu.VMEM((2, win_e), jnp.int32),
            pltpu.VMEM((2, win_e), jnp.float32),
            pltpu.SemaphoreType.DMA,
            pltpu.SemaphoreType.DMA,
            pltpu.SemaphoreType.DMA,
            pltpu.SemaphoreType.DMA,
        ],
    )
    def k(addr_hbm, vtab_hbm, out_hbm, vtab, awin, owin, si0, si1, so0, so1):
        wid = lax.axis_index("s") * NC + lax.axis_index("c")
        neuron = wid // 4
        b0 = (wid % 4) * chunk
        pltpu.sync_copy(vtab_hbm, vtab)
        isems = (si0, si1)
        osems = (so0, so1)
        idescs = [None, None]
        odescs = [None, None]
        idescs[0] = pltpu.async_copy(
            addr_hbm.at[neuron, pl.ds(b0, win_e)], awin.at[0], isems[0])
        for w in range(n_win):
            cur = w & 1
            if w + 1 < n_win:
                nxt = (w + 1) & 1
                idescs[nxt] = pltpu.async_copy(
                    addr_hbm.at[neuron, pl.ds(b0 + (w + 1) * win_e, win_e)],
                    awin.at[nxt], isems[nxt])
            idescs[cur].wait()
            if odescs[cur] is not None:
                odescs[cur].wait()

            def body(i, c, cur=cur):
                offs = [i * 128 + u * 16 for u in range(8)]
                idxv = [awin[cur, pl.ds(o, 16)] for o in offs]
                vals = [plsc.load_gather(vtab, [iv]) for iv in idxv]
                for o, v in zip(offs, vals):
                    owin[cur, pl.ds(o, 16)] = v
                return c

            lax.fori_loop(0, iters // 8, body, 0)
            odescs[cur] = pltpu.async_copy(
                owin.at[cur],
                out_hbm.at[neuron, pl.ds(b0 + w * win_e, win_e)],
                osems[cur])
        for d in odescs:
            if d is not None:
                d.wait()

    return k


def kernel(ctx_bits, mapping, target_vals, ram):
    del ram
    b_sz = ctx_bits.shape[0]

    powers = (1 << jnp.arange(N_BITS, dtype=jnp.int32))
    wt = jnp.sum(
        (mapping[:, :, None] == jnp.arange(TOTAL_BITS, dtype=jnp.int32)[None, None, :])
        * powers[None, :, None],
        axis=1,
    ).astype(jnp.float32)

    addr2 = _tc_addresses(wt, ctx_bits.T, b_sz, 16384)
    tv_flat = target_vals.T.reshape(-1)

    tabs = _sc_scatter_kernel(b_sz, 16384)(addr2)
    vtab = _sc_merge_kernel()(tabs, tv_flat)
    out2 = _sc_gather_kernel(b_sz, 16384)(addr2, vtab)
    return out2.T


# speedup vs baseline: 483.2785x; 1.1652x over previous
import functools

import jax
import jax.numpy as jnp
from jax import lax
from jax.experimental import pallas as pl
from jax.experimental.pallas import tpu as pltpu
from jax.experimental.pallas import tpu_sc as plsc

NUM_NEURONS = 8
N_BITS = 10
RAM_SIZE = 1 << N_BITS
TOTAL_BITS = 32

NC = 2
NS = 16
NPC = NUM_NEURONS // NC
QN = NS // NPC
SCTAB = NPC * RAM_SIZE


def _tc_addr_body(w_ref, ctx_ref, out_ref):
    x = ctx_ref[...].astype(jnp.float32)
    y = jnp.dot(w_ref[...], x, preferred_element_type=jnp.float32)
    row = lax.broadcasted_iota(jnp.int32, y.shape, 0)
    out_ref[...] = y.astype(jnp.int32) + row * RAM_SIZE


def _tc_addresses(wt, ctx_t, b_sz, cb):
    grid = (b_sz // cb,)
    return pl.pallas_call(
        _tc_addr_body,
        grid=grid,
        in_specs=[
            pl.BlockSpec((NUM_NEURONS, TOTAL_BITS), lambda i: (0, 0)),
            pl.BlockSpec((TOTAL_BITS, cb), lambda i: (0, i)),
        ],
        out_specs=pl.BlockSpec((NUM_NEURONS, cb), lambda i: (0, i)),
        out_shape=jax.ShapeDtypeStruct((NUM_NEURONS, b_sz), jnp.int32),
    )(wt, ctx_t)


def _sc_fused_kernel(b_sz, win_e):
    chunk = b_sz // QN
    n_win = chunk // win_e
    iters = win_e // 16
    e_mask = b_sz - 1
    e_shift = b_sz.bit_length() - 1
    mesh = plsc.VectorSubcoreMesh(core_axis_name="c", subcore_axis_name="s")

    @functools.partial(
        pl.kernel,
        out_type=jax.ShapeDtypeStruct((NUM_NEURONS, b_sz), jnp.float32),
        mesh=mesh,
        scratch_types=[
            pltpu.VMEM((2, win_e), jnp.int32),
            pltpu.VMEM((2, win_e), jnp.float32),
            pltpu.VMEM((RAM_SIZE,), jnp.int32),
            pltpu.VMEM((QN, 256), jnp.int32),
            pltpu.VMEM((2, 128), jnp.int32),
            pltpu.VMEM((2, 128), jnp.float32),
            pltpu.VMEM((SCTAB,), jnp.float32),
            pltpu.VMEM_SHARED((NS, RAM_SIZE), jnp.int32),
            pltpu.VMEM_SHARED((SCTAB,), jnp.float32),
            pltpu.SemaphoreType.DMA,
            pltpu.SemaphoreType.DMA,
            pltpu.SemaphoreType.DMA,
            pltpu.SemaphoreType.DMA,
            pltpu.SemaphoreType.DMA,
        ],
        compiler_params=pltpu.CompilerParams(needs_layout_passes=False),
    )
    def k(addr_hbm, tvp_hbm, out_hbm,
          awin, owin, tab, tloc, gidx, gval, vtab, stabs, svtab,
          si0, si1, so0, so1, sg):
        c = lax.axis_index("c")
        s = lax.axis_index("s")
        neuron = NPC * c + s // QN
        b0 = (s % QN) * chunk
        ebase = neuron * b_sz + b0
        cell0 = neuron * RAM_SIZE
        io16 = lax.iota(jnp.int32, 16)
        isems = (si0, si1)
        osems = (so0, so1)

        def init(i, cr):
            tab[pl.ds(i * 16, 16)] = jnp.full((16,), -1, jnp.int32)
            return cr

        lax.fori_loop(0, RAM_SIZE // 16, init, 0)

        descs = [None, None]
        descs[0] = pltpu.async_copy(
            addr_hbm.at[neuron, pl.ds(b0, win_e)], awin.at[0], isems[0])
        for w in range(n_win):
            cur = w & 1
            if w + 1 < n_win:
                nxt = (w + 1) & 1
                descs[nxt] = pltpu.async_copy(
                    addr_hbm.at[neuron, pl.ds(b0 + (w + 1) * win_e, win_e)],
                    awin.at[nxt], isems[nxt])
            descs[cur].wait()
            wbase = ebase + w * win_e

            def body(i, cr, wbase=wbase, cur=cur):
                offs = [i * 128 + u * 16 for u in range(8)]
                idxv = [awin[cur, pl.ds(o, 16)] - cell0 for o in offs]
                evs = [wbase + o + io16 for o in offs]
                for u in range(8):
                    plsc.store_scatter(tab, [idxv[u]], evs[u])
                return cr

            lax.fori_loop(0, iters // 8, body, 0)

        pltpu.sync_copy(tab, stabs.at[s])
        plsc.subcore_barrier()

        nl = s // QN
        col0 = (s % QN) * 256
        pltpu.sync_copy(
            stabs.at[pl.ds(nl * QN, QN), pl.ds(col0, 256)], tloc)
        for p in range(2):
            for g in range(8):
                o = p * 128 + g * 16
                m = tloc[0, pl.ds(o, 16)]
                for r in range(1, QN):
                    m = jnp.maximum(m, tloc[r, pl.ds(o, 16)])
                b = m & e_mask
                n = lax.shift_right_logical(m, e_shift)
                f = (lax.shift_right_logical(b, 7) * 1024
                     + n * 128 + (b & 127))
                gidx[p, pl.ds(g * 16, 16)] = jnp.where(m < 0, 0, f)
            pltpu.async_copy(tvp_hbm.at[gidx.at[p]], gval.at[p], sg).wait()
            pltpu.sync_copy(
                gval.at[p],
                svtab.at[pl.ds(nl * RAM_SIZE + col0 + p * 128, 128)])
        plsc.subcore_barrier()

        pltpu.sync_copy(svtab, vtab)
        sc_cell0 = c * SCTAB
        idescs = [None, None]
        odescs = [None, None]
        idescs[0] = pltpu.async_copy(
            addr_hbm.at[neuron, pl.ds(b0, win_e)], awin.at[0], isems[0])
        for w in range(n_win):
            cur = w & 1
            if w + 1 < n_win:
                nxt = (w + 1) & 1
                idescs[nxt] = pltpu.async_copy(
                    addr_hbm.at[neuron, pl.ds(b0 + (w + 1) * win_e, win_e)],
                    awin.at[nxt], isems[nxt])
            idescs[cur].wait()
            if odescs[cur] is not None:
                odescs[cur].wait()

            def body(i, cr, cur=cur):
                offs = [i * 128 + u * 16 for u in range(8)]
                idxv = [awin[cur, pl.ds(o, 16)] - sc_cell0 for o in offs]
                vals = [plsc.load_gather(vtab, [iv]) for iv in idxv]
                for o, v in zip(offs, vals):
                    owin[cur, pl.ds(o, 16)] = v
                return cr

            lax.fori_loop(0, iters // 8, body, 0)
            odescs[cur] = pltpu.async_copy(
                owin.at[cur],
                out_hbm.at[neuron, pl.ds(b0 + w * win_e, win_e)],
                osems[cur])
        for d in odescs:
            if d is not None:
                d.wait()

    return k


def kernel(ctx_bits, mapping, target_vals, ram):
    del ram
    b_sz = ctx_bits.shape[0]

    powers = (1 << jnp.arange(N_BITS, dtype=jnp.int32))
    wt = jnp.sum(
        (mapping[:, :, None] == jnp.arange(TOTAL_BITS, dtype=jnp.int32)[None, None, :])
        * powers[None, :, None],
        axis=1,
    ).astype(jnp.float32)

    addr2 = _tc_addresses(wt, ctx_bits.T, b_sz, 16384)

    tvp = target_vals.reshape(b_sz // 128, 128, NUM_NEURONS)
    tvp = tvp.transpose(0, 2, 1).reshape(-1)

    out2 = _sc_fused_kernel(b_sz, 16384)(addr2, tvp)
    return out2.T
